# baseline (speedup 1.0000x reference)
import jax
import jax.numpy as jnp
from jax import lax
from jax.experimental import pallas as pl
from jax.experimental.pallas import tpu as pltpu
from jax.experimental.pallas import tpu_sc as plsc

N = 10000
E = 320000
D = 128
G = 10
K = 3

NC, NS = 2, 16
NW = NC * NS
CH = 128
BASE = 80
NCHUNK = NW * BASE
EPAD = NCHUNK * CH - E
NA = N + 8
RPT = 624
RREM = N - NS * RPT
DEGW = 128

BN = 1000
NB = N // BN


def _zero_fill(buf, nrows, width):
    def row(i, _):
        for l in range(width // 16):
            buf[i, pl.ds(l * 16, 16)] = jnp.zeros((16,), jnp.float32)
        return 0
    lax.fori_loop(0, nrows, row, 0)


def _zero_acc(rows, acc, s):
    r0 = s * RPT
    for off, nr in ((0, 128), (128, 128), (256, 128), (384, 128), (512, 112)):
        pltpu.sync_copy(rows.at[pl.ds(0, nr)], acc.at[pl.ds(r0 + off, nr)])

    @pl.when(s == 0)
    def _():
        pltpu.sync_copy(rows.at[pl.ds(0, RREM)], acc.at[pl.ds(NS * RPT, RREM)])


def _publish(acc, out_hbm, c, s):
    r0 = s * RPT
    pltpu.sync_copy(acc.at[pl.ds(r0, RPT)], out_hbm.at[pl.ds(c * N + r0, RPT)])

    @pl.when(s == 0)
    def _():
        pltpu.sync_copy(acc.at[pl.ds(NS * RPT, RREM)],
                        out_hbm.at[pl.ds(c * N + NS * RPT, RREM)])


def _hop_body(src_hbm, dst_hbm, g_hbm, out_hbm,
              sidx, didx0, didx1, rows0, rows1, acc,
              semg0, semg1, semd0, semd1):
    c = lax.axis_index("c")
    s = lax.axis_index("s")
    wid = c * NS + s
    _zero_fill(rows0, CH, D)
    _zero_acc(rows0, acc, s)
    pltpu.sync_copy(src_hbm.at[pl.ds(wid * BASE, BASE)], sidx)
    plsc.subcore_barrier()
    e0 = wid * BASE * CH

    def chunk(j, _):
        pltpu.sync_copy(dst_hbm.at[pl.ds(e0 + j * CH, CH)], didx0)
        pltpu.async_copy(g_hbm.at[sidx.at[j]], rows0, semg0).wait()
        pltpu.sync_copy(rows0, acc.at[didx0], add=True)
        return 0
    lax.fori_loop(0, BASE, chunk, 0)

    plsc.subcore_barrier()
    _publish(acc, out_hbm, c, s)


def _deg_body(dst_hbm, out_hbm, didx0, didx1, rows, acc, semd0, semd1):
    c = lax.axis_index("c")
    s = lax.axis_index("s")
    wid = c * NS + s
    _zero_fill(rows, CH, DEGW)
    _zero_acc(rows, acc, s)
    def row(i, _):
        for l in range(DEGW // 16):
            rows[i, pl.ds(l * 16, 16)] = jnp.ones((16,), jnp.float32)
        return 0
    lax.fori_loop(0, CH, row, 0)
    plsc.subcore_barrier()
    e0 = wid * BASE * CH

    pltpu.async_copy(dst_hbm.at[pl.ds(e0, CH)], didx0, semd0)
    pltpu.async_copy(dst_hbm.at[pl.ds(e0 + CH, CH)], didx1, semd1)

    def pair(jj, _):
        j0 = 2 * jj
        pltpu.make_async_copy(dst_hbm.at[pl.ds(e0, CH)], didx0, semd0).wait()
        pltpu.sync_copy(rows, acc.at[didx0], add=True)

        @pl.when(jj < BASE // 2 - 1)
        def _():
            pltpu.async_copy(
                dst_hbm.at[pl.ds(e0 + (j0 + 2) * CH, CH)], didx0, semd0)

        pltpu.make_async_copy(dst_hbm.at[pl.ds(e0, CH)], didx1, semd1).wait()
        pltpu.sync_copy(rows, acc.at[didx1], add=True)

        @pl.when(jj < BASE // 2 - 1)
        def _():
            pltpu.async_copy(
                dst_hbm.at[pl.ds(e0 + (j0 + 3) * CH, CH)], didx1, semd1)
        return 0
    lax.fori_loop(0, BASE // 2, pair, 0)

    plsc.subcore_barrier()
    _publish(acc, out_hbm, c, s)


def _sc_mesh():
    return plsc.VectorSubcoreMesh(core_axis_name="c", subcore_axis_name="s",
                                  num_cores=NC, num_subcores=NS)


def _seg_sum(src, dst, g):
    return pl.kernel(
        _hop_body,
        out_type=jax.ShapeDtypeStruct((2 * N, D), jnp.float32),
        mesh=_sc_mesh(),
        scratch_types=[
            pltpu.VMEM((BASE, CH), jnp.int32),
            pltpu.VMEM((CH,), jnp.int32),
            pltpu.VMEM((CH,), jnp.int32),
            pltpu.VMEM((CH, D), jnp.float32),
            pltpu.VMEM((CH, D), jnp.float32),
            pltpu.VMEM_SHARED((NA, D), jnp.float32),
            pltpu.SemaphoreType.DMA,
            pltpu.SemaphoreType.DMA,
            pltpu.SemaphoreType.DMA,
            pltpu.SemaphoreType.DMA,
        ],
    )(src, dst, g)


def _degree(dst):
    return pl.kernel(
        _deg_body,
        out_type=jax.ShapeDtypeStruct((2 * N, DEGW), jnp.float32),
        mesh=_sc_mesh(),
        scratch_types=[
            pltpu.VMEM((CH,), jnp.int32),
            pltpu.VMEM((CH,), jnp.int32),
            pltpu.VMEM((CH, DEGW), jnp.float32),
            pltpu.VMEM_SHARED((NA, DEGW), jnp.float32),
            pltpu.SemaphoreType.DMA,
            pltpu.SemaphoreType.DMA,
        ],
    )(dst)


def _prep_body(p0, p1, x, dis_ref, g0_ref):
    deg = p0[:, 0:1] + p1[:, 0:1]
    dis = jnp.where(deg > 0, lax.rsqrt(jnp.maximum(deg, 1.0)), 0.0)
    dis_b = jnp.broadcast_to(dis, (BN, D))
    dis_ref[...] = dis_b
    g0_ref[...] = dis_b * x[...]


def _prep(degp, x):
    return pl.pallas_call(
        _prep_body,
        grid=(NB,),
        in_specs=[
            pl.BlockSpec((BN, DEGW), lambda i: (i, 0)),
            pl.BlockSpec((BN, DEGW), lambda i: (i + NB, 0)),
            pl.BlockSpec((BN, D), lambda i: (i, 0)),
        ],
        out_specs=[
            pl.BlockSpec((BN, D), lambda i: (i, 0)),
            pl.BlockSpec((BN, D), lambda i: (i, 0)),
        ],
        out_shape=[
            jax.ShapeDtypeStruct((N, D), jnp.float32),
            jax.ShapeDtypeStruct((N, D), jnp.float32),
        ],
    )(degp, degp, x)


def _combine_body(pa, pb, dis, h_ref, g_ref):
    h = dis[...] * (pa[...] + pb[...])
    h_ref[...] = h
    g_ref[...] = dis[...] * h


def _combine(p, dis_b):
    return pl.pallas_call(
        _combine_body,
        grid=(NB,),
        in_specs=[
            pl.BlockSpec((BN, D), lambda i: (i, 0)),
            pl.BlockSpec((BN, D), lambda i: (i + NB, 0)),
            pl.BlockSpec((BN, D), lambda i: (i, 0)),
        ],
        out_specs=[
            pl.BlockSpec((BN, D), lambda i: (i, 0)),
            pl.BlockSpec((BN, D), lambda i: (i, 0)),
        ],
        out_shape=[
            jax.ShapeDtypeStruct((N, D), jnp.float32),
            jax.ShapeDtypeStruct((N, D), jnp.float32),
        ],
    )(p, p, dis_b)


def _layer_out(x, h1, h2, pa, pb, dis, W, b):
    h3 = dis[...] * (pa[...] + pb[...])
    acc = jnp.dot(x[...], W[0], preferred_element_type=jnp.float32)
    acc = acc + jnp.dot(h1[...], W[1], preferred_element_type=jnp.float32)
    acc = acc + jnp.dot(h2[...], W[2], preferred_element_type=jnp.float32)
    acc = acc + jnp.dot(h3, W[3], preferred_element_type=jnp.float32)
    acc = acc + b[...]
    return jnp.where(acc >= 0, acc, 0.01 * acc)


def _finish1_body(x, h1, h2, pa, pb, dis, W, b, x2_ref, g_ref):
    x2 = _layer_out(x, h1, h2, pa, pb, dis, W, b)
    x2_ref[...] = x2
    g_ref[...] = dis[...] * x2


def _finish1(x, h1, h2, p3, dis_b, W, b):
    return pl.pallas_call(
        _finish1_body,
        grid=(NB,),
        in_specs=[
            pl.BlockSpec((BN, D), lambda i: (i, 0)),
            pl.BlockSpec((BN, D), lambda i: (i, 0)),
            pl.BlockSpec((BN, D), lambda i: (i, 0)),
            pl.BlockSpec((BN, D), lambda i: (i, 0)),
            pl.BlockSpec((BN, D), lambda i: (i + NB, 0)),
            pl.BlockSpec((BN, D), lambda i: (i, 0)),
            pl.BlockSpec((K + 1, D, D), lambda i: (0, 0, 0)),
            pl.BlockSpec((1, D), lambda i: (0, 0)),
        ],
        out_specs=[
            pl.BlockSpec((BN, D), lambda i: (i, 0)),
            pl.BlockSpec((BN, D), lambda i: (i, 0)),
        ],
        out_shape=[
            jax.ShapeDtypeStruct((N, D), jnp.float32),
            jax.ShapeDtypeStruct((N, D), jnp.float32),
        ],
    )(x, h1, h2, p3, p3, dis_b, W, b)


def _finish2_body(x, h1, h2, pa, pb, dis, W, b, wf, y_ref):
    i = pl.program_id(0)
    x3 = _layer_out(x, h1, h2, pa, pb, dis, W, b)

    @pl.when(i == 0)
    def _():
        y_ref[...] = jnp.zeros((16, D), jnp.float32)

    rows = [jnp.sum(x3 * wf[g], axis=0, keepdims=True) for g in range(G)]
    rows.append(jnp.zeros((16 - G, D), jnp.float32))
    y_ref[...] = y_ref[...] + jnp.concatenate(rows, axis=0)


def _finish2(x, h1, h2, p3, dis_b, W, b, wfr):
    return pl.pallas_call(
        _finish2_body,
        grid=(NB,),
        in_specs=[
            pl.BlockSpec((BN, D), lambda i: (i, 0)),
            pl.BlockSpec((BN, D), lambda i: (i, 0)),
            pl.BlockSpec((BN, D), lambda i: (i, 0)),
            pl.BlockSpec((BN, D), lambda i: (i, 0)),
            pl.BlockSpec((BN, D), lambda i: (i + NB, 0)),
            pl.BlockSpec((BN, D), lambda i: (i, 0)),
            pl.BlockSpec((K + 1, D, D), lambda i: (0, 0, 0)),
            pl.BlockSpec((1, D), lambda i: (0, 0)),
            pl.BlockSpec((G, BN, D), lambda i: (0, i, 0)),
        ],
        out_specs=pl.BlockSpec((16, D), lambda i: (0, 0)),
        out_shape=jax.ShapeDtypeStruct((16, D), jnp.float32),
    )(x, h1, h2, p3, p3, dis_b, W, b, wfr)


def kernel(x, edge_index, W1, b1, W2, b2, Wf, bf):
    src = jnp.concatenate(
        [edge_index[0], jnp.zeros((EPAD,), jnp.int32)]).reshape(NCHUNK, CH)
    dst = jnp.concatenate(
        [edge_index[1], jnp.full((EPAD,), N, jnp.int32)])
    wfr = Wf.reshape(G, N, D)
    b1r = b1.reshape(1, D)
    b2r = b2.reshape(1, D)

    degp = _degree(dst)
    dis_b, g = _prep(degp, x)

    h1, g = _combine(_seg_sum(src, dst, g), dis_b)
    h2, g = _combine(_seg_sum(src, dst, g), dis_b)
    p3 = _seg_sum(src, dst, g)
    x2, g = _finish1(x, h1, h2, p3, dis_b, W1, b1r)

    h1, g = _combine(_seg_sum(src, dst, g), dis_b)
    h2, g = _combine(_seg_sum(src, dst, g), dis_b)
    p3 = _seg_sum(src, dst, g)
    y16 = _finish2(x2, h1, h2, p3, dis_b, W2, b2r, wfr)

    return jnp.sum(y16[:G], axis=1) + bf


# speedup vs baseline: 1.2173x; 1.2173x over previous
import jax
import jax.numpy as jnp
from jax import lax
from jax.experimental import pallas as pl
from jax.experimental.pallas import tpu as pltpu
from jax.experimental.pallas import tpu_sc as plsc

N = 10000
E = 320000
D = 128
G = 10
K = 3

NC, NS = 2, 16
NW = NC * NS
CH = 128
BASE = 80
NCHUNK = NW * BASE
EPAD = NCHUNK * CH - E
NA = N + 8
RPT = 624
RREM = N - NS * RPT
DEGW = 128

BN = 1000
NB = N // BN


def _zero_fill(buf, nrows, width):
    def row(i, _):
        for l in range(width // 16):
            buf[i, pl.ds(l * 16, 16)] = jnp.zeros((16,), jnp.float32)
        return 0
    lax.fori_loop(0, nrows, row, 0)


def _zero_acc(rows, acc, s):
    r0 = s * RPT
    for off, nr in ((0, 128), (128, 128), (256, 128), (384, 128), (512, 112)):
        pltpu.sync_copy(rows.at[pl.ds(0, nr)], acc.at[pl.ds(r0 + off, nr)])

    @pl.when(s == 0)
    def _():
        pltpu.sync_copy(rows.at[pl.ds(0, RREM)], acc.at[pl.ds(NS * RPT, RREM)])


def _publish(acc, out_hbm, c, s):
    r0 = s * RPT
    pltpu.sync_copy(acc.at[pl.ds(r0, RPT)], out_hbm.at[pl.ds(c * N + r0, RPT)])

    @pl.when(s == 0)
    def _():
        pltpu.sync_copy(acc.at[pl.ds(NS * RPT, RREM)],
                        out_hbm.at[pl.ds(c * N + NS * RPT, RREM)])


def _hop_body(src_hbm, dst_hbm, g_hbm, out_hbm,
              sidx0, sidx1, didx0, didx1, rows0, rows1, acc,
              semg0, semg1):
    c = lax.axis_index("c")
    s = lax.axis_index("s")
    wid = c * NS + s
    _zero_fill(rows0, CH, D)
    _zero_acc(rows0, acc, s)
    plsc.subcore_barrier()
    e0 = wid * BASE * CH

    pltpu.sync_copy(src_hbm.at[pl.ds(e0, CH)], sidx0)
    pltpu.async_copy(g_hbm.at[sidx0], rows0, semg0)

    def pair(jj, _):
        j0 = 2 * jj
        pltpu.sync_copy(src_hbm.at[pl.ds(e0 + (j0 + 1) * CH, CH)], sidx1)
        pltpu.async_copy(g_hbm.at[sidx1], rows1, semg1)
        pltpu.sync_copy(dst_hbm.at[pl.ds(e0 + j0 * CH, CH)], didx0)
        pltpu.make_async_copy(g_hbm.at[sidx0], rows0, semg0).wait()
        pltpu.sync_copy(rows0, acc.at[didx0], add=True)

        @pl.when(jj < BASE // 2 - 1)
        def _():
            pltpu.sync_copy(src_hbm.at[pl.ds(e0 + (j0 + 2) * CH, CH)], sidx0)
            pltpu.async_copy(g_hbm.at[sidx0], rows0, semg0)
        pltpu.sync_copy(dst_hbm.at[pl.ds(e0 + (j0 + 1) * CH, CH)], didx1)
        pltpu.make_async_copy(g_hbm.at[sidx1], rows1, semg1).wait()
        pltpu.sync_copy(rows1, acc.at[didx1], add=True)
        return 0
    lax.fori_loop(0, BASE // 2, pair, 0)

    plsc.subcore_barrier()
    _publish(acc, out_hbm, c, s)


def _deg_body(dst_hbm, out_hbm, didx0, didx1, rows, acc, semd0, semd1):
    c = lax.axis_index("c")
    s = lax.axis_index("s")
    wid = c * NS + s
    _zero_fill(rows, CH, DEGW)
    _zero_acc(rows, acc, s)
    def row(i, _):
        for l in range(DEGW // 16):
            rows[i, pl.ds(l * 16, 16)] = jnp.ones((16,), jnp.float32)
        return 0
    lax.fori_loop(0, CH, row, 0)
    plsc.subcore_barrier()
    e0 = wid * BASE * CH

    pltpu.async_copy(dst_hbm.at[pl.ds(e0, CH)], didx0, semd0)
    pltpu.async_copy(dst_hbm.at[pl.ds(e0 + CH, CH)], didx1, semd1)

    def pair(jj, _):
        j0 = 2 * jj
        pltpu.make_async_copy(dst_hbm.at[pl.ds(e0, CH)], didx0, semd0).wait()
        pltpu.sync_copy(rows, acc.at[didx0], add=True)

        @pl.when(jj < BASE // 2 - 1)
        def _():
            pltpu.async_copy(
                dst_hbm.at[pl.ds(e0 + (j0 + 2) * CH, CH)], didx0, semd0)

        pltpu.make_async_copy(dst_hbm.at[pl.ds(e0, CH)], didx1, semd1).wait()
        pltpu.sync_copy(rows, acc.at[didx1], add=True)

        @pl.when(jj < BASE // 2 - 1)
        def _():
            pltpu.async_copy(
                dst_hbm.at[pl.ds(e0 + (j0 + 3) * CH, CH)], didx1, semd1)
        return 0
    lax.fori_loop(0, BASE // 2, pair, 0)

    plsc.subcore_barrier()
    _publish(acc, out_hbm, c, s)


def _sc_mesh():
    return plsc.VectorSubcoreMesh(core_axis_name="c", subcore_axis_name="s",
                                  num_cores=NC, num_subcores=NS)


def _seg_sum(src, dst, g):
    return pl.kernel(
        _hop_body,
        out_type=jax.ShapeDtypeStruct((2 * N, D), jnp.float32),
        mesh=_sc_mesh(),
        scratch_types=[
            pltpu.VMEM((CH,), jnp.int32),
            pltpu.VMEM((CH,), jnp.int32),
            pltpu.VMEM((CH,), jnp.int32),
            pltpu.VMEM((CH,), jnp.int32),
            pltpu.VMEM((CH, D), jnp.float32),
            pltpu.VMEM((CH, D), jnp.float32),
            pltpu.VMEM_SHARED((NA, D), jnp.float32),
            pltpu.SemaphoreType.DMA,
            pltpu.SemaphoreType.DMA,
        ],
    )(src, dst, g)


def _degree(dst):
    return pl.kernel(
        _deg_body,
        out_type=jax.ShapeDtypeStruct((2 * N, DEGW), jnp.float32),
        mesh=_sc_mesh(),
        scratch_types=[
            pltpu.VMEM((CH,), jnp.int32),
            pltpu.VMEM((CH,), jnp.int32),
            pltpu.VMEM((CH, DEGW), jnp.float32),
            pltpu.VMEM_SHARED((NA, DEGW), jnp.float32),
            pltpu.SemaphoreType.DMA,
            pltpu.SemaphoreType.DMA,
        ],
    )(dst)


def _prep_body(p0, p1, x, dis_ref, g0_ref):
    deg = p0[:, 0:1] + p1[:, 0:1]
    dis = jnp.where(deg > 0, lax.rsqrt(jnp.maximum(deg, 1.0)), 0.0)
    dis_b = jnp.broadcast_to(dis, (BN, D))
    dis_ref[...] = dis_b
    g0_ref[...] = dis_b * x[...]


def _prep(degp, x):
    return pl.pallas_call(
        _prep_body,
        grid=(NB,),
        in_specs=[
            pl.BlockSpec((BN, DEGW), lambda i: (i, 0)),
            pl.BlockSpec((BN, DEGW), lambda i: (i + NB, 0)),
            pl.BlockSpec((BN, D), lambda i: (i, 0)),
        ],
        out_specs=[
            pl.BlockSpec((BN, D), lambda i: (i, 0)),
            pl.BlockSpec((BN, D), lambda i: (i, 0)),
        ],
        out_shape=[
            jax.ShapeDtypeStruct((N, D), jnp.float32),
            jax.ShapeDtypeStruct((N, D), jnp.float32),
        ],
    )(degp, degp, x)


def _combine_body(pa, pb, dis, h_ref, g_ref):
    h = dis[...] * (pa[...] + pb[...])
    h_ref[...] = h
    g_ref[...] = dis[...] * h


def _combine(p, dis_b):
    return pl.pallas_call(
        _combine_body,
        grid=(NB,),
        in_specs=[
            pl.BlockSpec((BN, D), lambda i: (i, 0)),
            pl.BlockSpec((BN, D), lambda i: (i + NB, 0)),
            pl.BlockSpec((BN, D), lambda i: (i, 0)),
        ],
        out_specs=[
            pl.BlockSpec((BN, D), lambda i: (i, 0)),
            pl.BlockSpec((BN, D), lambda i: (i, 0)),
        ],
        out_shape=[
            jax.ShapeDtypeStruct((N, D), jnp.float32),
            jax.ShapeDtypeStruct((N, D), jnp.float32),
        ],
    )(p, p, dis_b)


def _layer_out(x, h1, h2, pa, pb, dis, W, b):
    h3 = dis[...] * (pa[...] + pb[...])
    acc = jnp.dot(x[...], W[0], preferred_element_type=jnp.float32)
    acc = acc + jnp.dot(h1[...], W[1], preferred_element_type=jnp.float32)
    acc = acc + jnp.dot(h2[...], W[2], preferred_element_type=jnp.float32)
    acc = acc + jnp.dot(h3, W[3], preferred_element_type=jnp.float32)
    acc = acc + b[...]
    return jnp.where(acc >= 0, acc, 0.01 * acc)


def _finish1_body(x, h1, h2, pa, pb, dis, W, b, x2_ref, g_ref):
    x2 = _layer_out(x, h1, h2, pa, pb, dis, W, b)
    x2_ref[...] = x2
    g_ref[...] = dis[...] * x2


def _finish1(x, h1, h2, p3, dis_b, W, b):
    return pl.pallas_call(
        _finish1_body,
        grid=(NB,),
        in_specs=[
            pl.BlockSpec((BN, D), lambda i: (i, 0)),
            pl.BlockSpec((BN, D), lambda i: (i, 0)),
            pl.BlockSpec((BN, D), lambda i: (i, 0)),
            pl.BlockSpec((BN, D), lambda i: (i, 0)),
            pl.BlockSpec((BN, D), lambda i: (i + NB, 0)),
            pl.BlockSpec((BN, D), lambda i: (i, 0)),
            pl.BlockSpec((K + 1, D, D), lambda i: (0, 0, 0)),
            pl.BlockSpec((1, D), lambda i: (0, 0)),
        ],
        out_specs=[
            pl.BlockSpec((BN, D), lambda i: (i, 0)),
            pl.BlockSpec((BN, D), lambda i: (i, 0)),
        ],
        out_shape=[
            jax.ShapeDtypeStruct((N, D), jnp.float32),
            jax.ShapeDtypeStruct((N, D), jnp.float32),
        ],
    )(x, h1, h2, p3, p3, dis_b, W, b)


def _finish2_body(x, h1, h2, pa, pb, dis, W, b, wf, y_ref):
    i = pl.program_id(0)
    x3 = _layer_out(x, h1, h2, pa, pb, dis, W, b)

    @pl.when(i == 0)
    def _():
        y_ref[...] = jnp.zeros((16, D), jnp.float32)

    rows = [jnp.sum(x3 * wf[g], axis=0, keepdims=True) for g in range(G)]
    rows.append(jnp.zeros((16 - G, D), jnp.float32))
    y_ref[...] = y_ref[...] + jnp.concatenate(rows, axis=0)


def _finish2(x, h1, h2, p3, dis_b, W, b, wfr):
    return pl.pallas_call(
        _finish2_body,
        grid=(NB,),
        in_specs=[
            pl.BlockSpec((BN, D), lambda i: (i, 0)),
            pl.BlockSpec((BN, D), lambda i: (i, 0)),
            pl.BlockSpec((BN, D), lambda i: (i, 0)),
            pl.BlockSpec((BN, D), lambda i: (i, 0)),
            pl.BlockSpec((BN, D), lambda i: (i + NB, 0)),
            pl.BlockSpec((BN, D), lambda i: (i, 0)),
            pl.BlockSpec((K + 1, D, D), lambda i: (0, 0, 0)),
            pl.BlockSpec((1, D), lambda i: (0, 0)),
            pl.BlockSpec((G, BN, D), lambda i: (0, i, 0)),
        ],
        out_specs=pl.BlockSpec((16, D), lambda i: (0, 0)),
        out_shape=jax.ShapeDtypeStruct((16, D), jnp.float32),
    )(x, h1, h2, p3, p3, dis_b, W, b, wfr)


def kernel(x, edge_index, W1, b1, W2, b2, Wf, bf):
    src = jnp.concatenate(
        [edge_index[0], jnp.zeros((EPAD,), jnp.int32)])
    dst = jnp.concatenate(
        [edge_index[1], jnp.full((EPAD,), N, jnp.int32)])
    wfr = Wf.reshape(G, N, D)
    b1r = b1.reshape(1, D)
    b2r = b2.reshape(1, D)

    degp = _degree(dst)
    dis_b, g = _prep(degp, x)

    h1, g = _combine(_seg_sum(src, dst, g), dis_b)
    h2, g = _combine(_seg_sum(src, dst, g), dis_b)
    p3 = _seg_sum(src, dst, g)
    x2, g = _finish1(x, h1, h2, p3, dis_b, W1, b1r)

    h1, g = _combine(_seg_sum(src, dst, g), dis_b)
    h2, g = _combine(_seg_sum(src, dst, g), dis_b)
    p3 = _seg_sum(src, dst, g)
    y16 = _finish2(x2, h1, h2, p3, dis_b, W2, b2r, wfr)

    return jnp.sum(y16[:G], axis=1) + bf


# speedup vs baseline: 1.2232x; 1.0048x over previous
import jax
import jax.numpy as jnp
from jax import lax
from jax.experimental import pallas as pl
from jax.experimental.pallas import tpu as pltpu
from jax.experimental.pallas import tpu_sc as plsc

N = 10000
E = 320000
D = 128
G = 10
K = 3

NC, NS = 2, 16
NW = NC * NS
CH = 128
BASE = 80
NCHUNK = NW * BASE
EPAD = NCHUNK * CH - E
NA = N + CH
RPT = 624
RREM = N - NS * RPT
DEGW = 128

BN = 1000
NB = N // BN


def _zero_fill(buf, nrows, width):
    def row(i, _):
        for l in range(width // 16):
            buf[i, pl.ds(l * 16, 16)] = jnp.zeros((16,), jnp.float32)
        return 0
    lax.fori_loop(0, nrows, row, 0)


def _zero_acc(rows, acc, s):
    r0 = s * RPT
    for off, nr in ((0, 128), (128, 128), (256, 128), (384, 128), (512, 112)):
        pltpu.sync_copy(rows.at[pl.ds(0, nr)], acc.at[pl.ds(r0 + off, nr)])

    @pl.when(s == 0)
    def _():
        pltpu.sync_copy(rows.at[pl.ds(0, RREM)], acc.at[pl.ds(NS * RPT, RREM)])


def _publish(acc, out_hbm, c, s):
    r0 = s * RPT
    pltpu.sync_copy(acc.at[pl.ds(r0, RPT)], out_hbm.at[pl.ds(c * N + r0, RPT)])

    @pl.when(s == 0)
    def _():
        pltpu.sync_copy(acc.at[pl.ds(NS * RPT, RREM)],
                        out_hbm.at[pl.ds(c * N + NS * RPT, RREM)])


def _hop_body(src_hbm, dst_hbm, g_hbm, out_hbm,
              sidx0, sidx1, didx0, didx1, rows0, rows1, acc,
              semg0, semg1):
    c = lax.axis_index("c")
    s = lax.axis_index("s")
    wid = c * NS + s
    _zero_fill(rows0, CH, D)
    _zero_acc(rows0, acc, s)
    plsc.subcore_barrier()
    e0 = wid * BASE * CH

    pltpu.sync_copy(src_hbm.at[pl.ds(e0, CH)], sidx0)
    pltpu.async_copy(g_hbm.at[sidx0], rows0, semg0)

    def pair(jj, _):
        j0 = 2 * jj
        pltpu.sync_copy(src_hbm.at[pl.ds(e0 + (j0 + 1) * CH, CH)], sidx1)
        pltpu.async_copy(g_hbm.at[sidx1], rows1, semg1)
        pltpu.sync_copy(dst_hbm.at[pl.ds(e0 + j0 * CH, CH)], didx0)
        pltpu.make_async_copy(g_hbm.at[sidx0], rows0, semg0).wait()
        pltpu.sync_copy(rows0, acc.at[didx0], add=True)

        @pl.when(jj < BASE // 2 - 1)
        def _():
            pltpu.sync_copy(src_hbm.at[pl.ds(e0 + (j0 + 2) * CH, CH)], sidx0)
            pltpu.async_copy(g_hbm.at[sidx0], rows0, semg0)
        pltpu.sync_copy(dst_hbm.at[pl.ds(e0 + (j0 + 1) * CH, CH)], didx1)
        pltpu.make_async_copy(g_hbm.at[sidx1], rows1, semg1).wait()
        pltpu.sync_copy(rows1, acc.at[didx1], add=True)
        return 0
    lax.fori_loop(0, BASE // 2, pair, 0)

    plsc.subcore_barrier()
    _publish(acc, out_hbm, c, s)


def _deg_body(dst_hbm, out_hbm, didx0, didx1, rows, acc, semd0, semd1):
    c = lax.axis_index("c")
    s = lax.axis_index("s")
    wid = c * NS + s
    _zero_fill(rows, CH, DEGW)
    _zero_acc(rows, acc, s)
    def row(i, _):
        for l in range(DEGW // 16):
            rows[i, pl.ds(l * 16, 16)] = jnp.ones((16,), jnp.float32)
        return 0
    lax.fori_loop(0, CH, row, 0)
    plsc.subcore_barrier()
    e0 = wid * BASE * CH

    pltpu.async_copy(dst_hbm.at[pl.ds(e0, CH)], didx0, semd0)
    pltpu.async_copy(dst_hbm.at[pl.ds(e0 + CH, CH)], didx1, semd1)

    def pair(jj, _):
        j0 = 2 * jj
        pltpu.make_async_copy(dst_hbm.at[pl.ds(e0, CH)], didx0, semd0).wait()
        pltpu.sync_copy(rows, acc.at[didx0], add=True)

        @pl.when(jj < BASE // 2 - 1)
        def _():
            pltpu.async_copy(
                dst_hbm.at[pl.ds(e0 + (j0 + 2) * CH, CH)], didx0, semd0)

        pltpu.make_async_copy(dst_hbm.at[pl.ds(e0, CH)], didx1, semd1).wait()
        pltpu.sync_copy(rows, acc.at[didx1], add=True)

        @pl.when(jj < BASE // 2 - 1)
        def _():
            pltpu.async_copy(
                dst_hbm.at[pl.ds(e0 + (j0 + 3) * CH, CH)], didx1, semd1)
        return 0
    lax.fori_loop(0, BASE // 2, pair, 0)

    plsc.subcore_barrier()
    _publish(acc, out_hbm, c, s)


def _sc_mesh():
    return plsc.VectorSubcoreMesh(core_axis_name="c", subcore_axis_name="s",
                                  num_cores=NC, num_subcores=NS)


def _seg_sum(src, dst, g):
    return pl.kernel(
        _hop_body,
        out_type=jax.ShapeDtypeStruct((2 * N, D), jnp.float32),
        mesh=_sc_mesh(),
        scratch_types=[
            pltpu.VMEM((CH,), jnp.int32),
            pltpu.VMEM((CH,), jnp.int32),
            pltpu.VMEM((CH,), jnp.int32),
            pltpu.VMEM((CH,), jnp.int32),
            pltpu.VMEM((CH, D), jnp.float32),
            pltpu.VMEM((CH, D), jnp.float32),
            pltpu.VMEM_SHARED((NA, D), jnp.float32),
            pltpu.SemaphoreType.DMA,
            pltpu.SemaphoreType.DMA,
        ],
    )(src, dst, g)


def _degree(dst):
    return pl.kernel(
        _deg_body,
        out_type=jax.ShapeDtypeStruct((2 * N, DEGW), jnp.float32),
        mesh=_sc_mesh(),
        scratch_types=[
            pltpu.VMEM((CH,), jnp.int32),
            pltpu.VMEM((CH,), jnp.int32),
            pltpu.VMEM((CH, DEGW), jnp.float32),
            pltpu.VMEM_SHARED((NA, DEGW), jnp.float32),
            pltpu.SemaphoreType.DMA,
            pltpu.SemaphoreType.DMA,
        ],
    )(dst)


def _prep_body(p0, p1, x, dis_ref, g0_ref):
    deg = p0[:, 0:1] + p1[:, 0:1]
    dis = jnp.where(deg > 0, lax.rsqrt(jnp.maximum(deg, 1.0)), 0.0)
    dis_b = jnp.broadcast_to(dis, (BN, D))
    dis_ref[...] = dis_b
    g0_ref[...] = dis_b * x[...]


def _prep(degp, x):
    return pl.pallas_call(
        _prep_body,
        grid=(NB,),
        in_specs=[
            pl.BlockSpec((BN, DEGW), lambda i: (i, 0)),
            pl.BlockSpec((BN, DEGW), lambda i: (i + NB, 0)),
            pl.BlockSpec((BN, D), lambda i: (i, 0)),
        ],
        out_specs=[
            pl.BlockSpec((BN, D), lambda i: (i, 0)),
            pl.BlockSpec((BN, D), lambda i: (i, 0)),
        ],
        out_shape=[
            jax.ShapeDtypeStruct((N, D), jnp.float32),
            jax.ShapeDtypeStruct((N, D), jnp.float32),
        ],
    )(degp, degp, x)


def _combine_body(pa, pb, dis, h_ref, g_ref):
    h = dis[...] * (pa[...] + pb[...])
    h_ref[...] = h
    g_ref[...] = dis[...] * h


def _combine(p, dis_b):
    return pl.pallas_call(
        _combine_body,
        grid=(NB,),
        in_specs=[
            pl.BlockSpec((BN, D), lambda i: (i, 0)),
            pl.BlockSpec((BN, D), lambda i: (i + NB, 0)),
            pl.BlockSpec((BN, D), lambda i: (i, 0)),
        ],
        out_specs=[
            pl.BlockSpec((BN, D), lambda i: (i, 0)),
            pl.BlockSpec((BN, D), lambda i: (i, 0)),
        ],
        out_shape=[
            jax.ShapeDtypeStruct((N, D), jnp.float32),
            jax.ShapeDtypeStruct((N, D), jnp.float32),
        ],
    )(p, p, dis_b)


def _layer_out(x, h1, h2, pa, pb, dis, W, b):
    h3 = dis[...] * (pa[...] + pb[...])
    acc = jnp.dot(x[...], W[0], preferred_element_type=jnp.float32)
    acc = acc + jnp.dot(h1[...], W[1], preferred_element_type=jnp.float32)
    acc = acc + jnp.dot(h2[...], W[2], preferred_element_type=jnp.float32)
    acc = acc + jnp.dot(h3, W[3], preferred_element_type=jnp.float32)
    acc = acc + b[...]
    return jnp.where(acc >= 0, acc, 0.01 * acc)


def _finish1_body(x, h1, h2, pa, pb, dis, W, b, x2_ref, g_ref):
    x2 = _layer_out(x, h1, h2, pa, pb, dis, W, b)
    x2_ref[...] = x2
    g_ref[...] = dis[...] * x2


def _finish1(x, h1, h2, p3, dis_b, W, b):
    return pl.pallas_call(
        _finish1_body,
        grid=(NB,),
        in_specs=[
            pl.BlockSpec((BN, D), lambda i: (i, 0)),
            pl.BlockSpec((BN, D), lambda i: (i, 0)),
            pl.BlockSpec((BN, D), lambda i: (i, 0)),
            pl.BlockSpec((BN, D), lambda i: (i, 0)),
            pl.BlockSpec((BN, D), lambda i: (i + NB, 0)),
            pl.BlockSpec((BN, D), lambda i: (i, 0)),
            pl.BlockSpec((K + 1, D, D), lambda i: (0, 0, 0)),
            pl.BlockSpec((1, D), lambda i: (0, 0)),
        ],
        out_specs=[
            pl.BlockSpec((BN, D), lambda i: (i, 0)),
            pl.BlockSpec((BN, D), lambda i: (i, 0)),
        ],
        out_shape=[
            jax.ShapeDtypeStruct((N, D), jnp.float32),
            jax.ShapeDtypeStruct((N, D), jnp.float32),
        ],
    )(x, h1, h2, p3, p3, dis_b, W, b)


def _finish2_body(x, h1, h2, pa, pb, dis, W, b, wf, y_ref):
    i = pl.program_id(0)
    x3 = _layer_out(x, h1, h2, pa, pb, dis, W, b)

    @pl.when(i == 0)
    def _():
        y_ref[...] = jnp.zeros((16, D), jnp.float32)

    rows = [jnp.sum(x3 * wf[g], axis=0, keepdims=True) for g in range(G)]
    rows.append(jnp.zeros((16 - G, D), jnp.float32))
    y_ref[...] = y_ref[...] + jnp.concatenate(rows, axis=0)


def _finish2(x, h1, h2, p3, dis_b, W, b, wfr):
    return pl.pallas_call(
        _finish2_body,
        grid=(NB,),
        in_specs=[
            pl.BlockSpec((BN, D), lambda i: (i, 0)),
            pl.BlockSpec((BN, D), lambda i: (i, 0)),
            pl.BlockSpec((BN, D), lambda i: (i, 0)),
            pl.BlockSpec((BN, D), lambda i: (i, 0)),
            pl.BlockSpec((BN, D), lambda i: (i + NB, 0)),
            pl.BlockSpec((BN, D), lambda i: (i, 0)),
            pl.BlockSpec((K + 1, D, D), lambda i: (0, 0, 0)),
            pl.BlockSpec((1, D), lambda i: (0, 0)),
            pl.BlockSpec((G, BN, D), lambda i: (0, i, 0)),
        ],
        out_specs=pl.BlockSpec((16, D), lambda i: (0, 0)),
        out_shape=jax.ShapeDtypeStruct((16, D), jnp.float32),
    )(x, h1, h2, p3, p3, dis_b, W, b, wfr)


def kernel(x, edge_index, W1, b1, W2, b2, Wf, bf):
    src = jnp.concatenate(
        [edge_index[0], jnp.zeros((EPAD,), jnp.int32)])
    dst = jnp.concatenate(
        [edge_index[1], N + (jnp.arange(EPAD, dtype=jnp.int32) % CH)])
    wfr = Wf.reshape(G, N, D)
    b1r = b1.reshape(1, D)
    b2r = b2.reshape(1, D)

    degp = _degree(dst)
    dis_b, g = _prep(degp, x)

    h1, g = _combine(_seg_sum(src, dst, g), dis_b)
    h2, g = _combine(_seg_sum(src, dst, g), dis_b)
    p3 = _seg_sum(src, dst, g)
    x2, g = _finish1(x, h1, h2, p3, dis_b, W1, b1r)

    h1, g = _combine(_seg_sum(src, dst, g), dis_b)
    h2, g = _combine(_seg_sum(src, dst, g), dis_b)
    p3 = _seg_sum(src, dst, g)
    y16 = _finish2(x2, h1, h2, p3, dis_b, W2, b2r, wfr)

    return jnp.sum(y16[:G], axis=1) + bf


# speedup vs baseline: 2.7381x; 2.2386x over previous
import jax
import jax.numpy as jnp
from jax import lax
from jax.experimental import pallas as pl
from jax.experimental.pallas import tpu as pltpu
from jax.experimental.pallas import tpu_sc as plsc

N = 10000
E = 320000
D = 128
G = 10
K = 3

NC, NS = 2, 16
NW = NC * NS
CH = 128
NCHUNK = E // CH
BASE = NCHUNK // NW
EXTRA = NCHUNK - BASE * NW
NA = N
RPT = 624
RREM = N - NS * RPT
DEGW = 128

BN = 1000
NB = N // BN


def _zero_fill(buf, nrows, width):
    def row(i, _):
        for l in range(width // 16):
            buf[i, pl.ds(l * 16, 16)] = jnp.zeros((16,), jnp.float32)
        return 0
    lax.fori_loop(0, nrows, row, 0)


def _zero_acc(rows, acc, s):
    r0 = s * RPT
    for off, nr in ((0, 128), (128, 128), (256, 128), (384, 128), (512, 112)):
        pltpu.sync_copy(rows.at[pl.ds(0, nr)], acc.at[pl.ds(r0 + off, nr)])

    @pl.when(s == 0)
    def _():
        pltpu.sync_copy(rows.at[pl.ds(0, RREM)], acc.at[pl.ds(NS * RPT, RREM)])


def _publish(acc, out_hbm, c, s):
    r0 = s * RPT
    pltpu.sync_copy(acc.at[pl.ds(r0, RPT)], out_hbm.at[pl.ds(c * N + r0, RPT)])

    @pl.when(s == 0)
    def _():
        pltpu.sync_copy(acc.at[pl.ds(NS * RPT, RREM)],
                        out_hbm.at[pl.ds(c * N + NS * RPT, RREM)])


def _hop_body(src_hbm, dst_hbm, g_hbm, out_hbm,
              sidx0, sidx1, didx0, didx1, rows0, rows1, acc,
              semg0, semg1):
    c = lax.axis_index("c")
    s = lax.axis_index("s")
    wid = c * NS + s
    _zero_fill(rows0, CH, D)
    _zero_acc(rows0, acc, s)
    plsc.subcore_barrier()
    e0 = wid * BASE * CH

    pltpu.sync_copy(src_hbm.at[pl.ds(e0, CH)], sidx0)
    pltpu.async_copy(g_hbm.at[sidx0], rows0, semg0)

    def pair(jj, _):
        j0 = 2 * jj
        pltpu.sync_copy(src_hbm.at[pl.ds(e0 + (j0 + 1) * CH, CH)], sidx1)
        pltpu.async_copy(g_hbm.at[sidx1], rows1, semg1)
        pltpu.sync_copy(dst_hbm.at[pl.ds(e0 + j0 * CH, CH)], didx0)
        pltpu.make_async_copy(g_hbm.at[sidx0], rows0, semg0).wait()
        pltpu.sync_copy(rows0, acc.at[didx0], add=True)

        @pl.when(jj < BASE // 2 - 1)
        def _():
            pltpu.sync_copy(src_hbm.at[pl.ds(e0 + (j0 + 2) * CH, CH)], sidx0)
            pltpu.async_copy(g_hbm.at[sidx0], rows0, semg0)
        pltpu.sync_copy(dst_hbm.at[pl.ds(e0 + (j0 + 1) * CH, CH)], didx1)
        pltpu.make_async_copy(g_hbm.at[sidx1], rows1, semg1).wait()
        pltpu.sync_copy(rows1, acc.at[didx1], add=True)
        return 0
    lax.fori_loop(0, BASE // 2, pair, 0)

    @pl.when(wid < EXTRA)
    def _():
        ex = (NW * BASE + wid) * CH
        pltpu.sync_copy(src_hbm.at[pl.ds(ex, CH)], sidx0)
        pltpu.sync_copy(dst_hbm.at[pl.ds(ex, CH)], didx0)
        pltpu.async_copy(g_hbm.at[sidx0], rows0, semg0).wait()
        pltpu.sync_copy(rows0, acc.at[didx0], add=True)

    plsc.subcore_barrier()
    _publish(acc, out_hbm, c, s)


def _deg_body(dst_hbm, out_hbm, didx0, didx1, rows, acc, semd0, semd1):
    c = lax.axis_index("c")
    s = lax.axis_index("s")
    wid = c * NS + s
    _zero_fill(rows, CH, DEGW)
    _zero_acc(rows, acc, s)
    def row(i, _):
        for l in range(DEGW // 16):
            rows[i, pl.ds(l * 16, 16)] = jnp.ones((16,), jnp.float32)
        return 0
    lax.fori_loop(0, CH, row, 0)
    plsc.subcore_barrier()
    e0 = wid * BASE * CH

    pltpu.async_copy(dst_hbm.at[pl.ds(e0, CH)], didx0, semd0)
    pltpu.async_copy(dst_hbm.at[pl.ds(e0 + CH, CH)], didx1, semd1)

    def pair(jj, _):
        j0 = 2 * jj
        pltpu.make_async_copy(dst_hbm.at[pl.ds(e0, CH)], didx0, semd0).wait()
        pltpu.sync_copy(rows, acc.at[didx0], add=True)

        @pl.when(jj < BASE // 2 - 1)
        def _():
            pltpu.async_copy(
                dst_hbm.at[pl.ds(e0 + (j0 + 2) * CH, CH)], didx0, semd0)

        pltpu.make_async_copy(dst_hbm.at[pl.ds(e0, CH)], didx1, semd1).wait()
        pltpu.sync_copy(rows, acc.at[didx1], add=True)

        @pl.when(jj < BASE // 2 - 1)
        def _():
            pltpu.async_copy(
                dst_hbm.at[pl.ds(e0 + (j0 + 3) * CH, CH)], didx1, semd1)
        return 0
    lax.fori_loop(0, BASE // 2, pair, 0)

    @pl.when(wid < EXTRA)
    def _():
        ex = (NW * BASE + wid) * CH
        pltpu.sync_copy(dst_hbm.at[pl.ds(ex, CH)], didx0)
        pltpu.sync_copy(rows, acc.at[didx0], add=True)

    plsc.subcore_barrier()
    _publish(acc, out_hbm, c, s)


def _sc_mesh():
    return plsc.VectorSubcoreMesh(core_axis_name="c", subcore_axis_name="s",
                                  num_cores=NC, num_subcores=NS)


def _seg_sum(src, dst, g):
    return pl.kernel(
        _hop_body,
        out_type=jax.ShapeDtypeStruct((2 * N, D), jnp.float32),
        mesh=_sc_mesh(),
        scratch_types=[
            pltpu.VMEM((CH,), jnp.int32),
            pltpu.VMEM((CH,), jnp.int32),
            pltpu.VMEM((CH,), jnp.int32),
            pltpu.VMEM((CH,), jnp.int32),
            pltpu.VMEM((CH, D), jnp.float32),
            pltpu.VMEM((CH, D), jnp.float32),
            pltpu.VMEM_SHARED((NA, D), jnp.float32),
            pltpu.SemaphoreType.DMA,
            pltpu.SemaphoreType.DMA,
        ],
    )(src, dst, g)


def _degree(dst):
    return pl.kernel(
        _deg_body,
        out_type=jax.ShapeDtypeStruct((2 * N, DEGW), jnp.float32),
        mesh=_sc_mesh(),
        scratch_types=[
            pltpu.VMEM((CH,), jnp.int32),
            pltpu.VMEM((CH,), jnp.int32),
            pltpu.VMEM((CH, DEGW), jnp.float32),
            pltpu.VMEM_SHARED((NA, DEGW), jnp.float32),
            pltpu.SemaphoreType.DMA,
            pltpu.SemaphoreType.DMA,
        ],
    )(dst)


def _prep_body(p0, p1, x, dis_ref, g0_ref):
    deg = p0[:, 0:1] + p1[:, 0:1]
    dis = jnp.where(deg > 0, lax.rsqrt(jnp.maximum(deg, 1.0)), 0.0)
    dis_b = jnp.broadcast_to(dis, (BN, D))
    dis_ref[...] = dis_b
    g0_ref[...] = dis_b * x[...]


def _prep(degp, x):
    return pl.pallas_call(
        _prep_body,
        grid=(NB,),
        in_specs=[
            pl.BlockSpec((BN, DEGW), lambda i: (i, 0)),
            pl.BlockSpec((BN, DEGW), lambda i: (i + NB, 0)),
            pl.BlockSpec((BN, D), lambda i: (i, 0)),
        ],
        out_specs=[
            pl.BlockSpec((BN, D), lambda i: (i, 0)),
            pl.BlockSpec((BN, D), lambda i: (i, 0)),
        ],
        out_shape=[
            jax.ShapeDtypeStruct((N, D), jnp.float32),
            jax.ShapeDtypeStruct((N, D), jnp.float32),
        ],
    )(degp, degp, x)


def _combine_body(pa, pb, dis, h_ref, g_ref):
    h = dis[...] * (pa[...] + pb[...])
    h_ref[...] = h
    g_ref[...] = dis[...] * h


def _combine(p, dis_b):
    return pl.pallas_call(
        _combine_body,
        grid=(NB,),
        in_specs=[
            pl.BlockSpec((BN, D), lambda i: (i, 0)),
            pl.BlockSpec((BN, D), lambda i: (i + NB, 0)),
            pl.BlockSpec((BN, D), lambda i: (i, 0)),
        ],
        out_specs=[
            pl.BlockSpec((BN, D), lambda i: (i, 0)),
            pl.BlockSpec((BN, D), lambda i: (i, 0)),
        ],
        out_shape=[
            jax.ShapeDtypeStruct((N, D), jnp.float32),
            jax.ShapeDtypeStruct((N, D), jnp.float32),
        ],
    )(p, p, dis_b)


def _layer_out(x, h1, h2, pa, pb, dis, W, b):
    h3 = dis[...] * (pa[...] + pb[...])
    acc = jnp.dot(x[...], W[0], preferred_element_type=jnp.float32)
    acc = acc + jnp.dot(h1[...], W[1], preferred_element_type=jnp.float32)
    acc = acc + jnp.dot(h2[...], W[2], preferred_element_type=jnp.float32)
    acc = acc + jnp.dot(h3, W[3], preferred_element_type=jnp.float32)
    acc = acc + b[...]
    return jnp.where(acc >= 0, acc, 0.01 * acc)


def _finish1_body(x, h1, h2, pa, pb, dis, W, b, x2_ref, g_ref):
    x2 = _layer_out(x, h1, h2, pa, pb, dis, W, b)
    x2_ref[...] = x2
    g_ref[...] = dis[...] * x2


def _finish1(x, h1, h2, p3, dis_b, W, b):
    return pl.pallas_call(
        _finish1_body,
        grid=(NB,),
        in_specs=[
            pl.BlockSpec((BN, D), lambda i: (i, 0)),
            pl.BlockSpec((BN, D), lambda i: (i, 0)),
            pl.BlockSpec((BN, D), lambda i: (i, 0)),
            pl.BlockSpec((BN, D), lambda i: (i, 0)),
            pl.BlockSpec((BN, D), lambda i: (i + NB, 0)),
            pl.BlockSpec((BN, D), lambda i: (i, 0)),
            pl.BlockSpec((K + 1, D, D), lambda i: (0, 0, 0)),
            pl.BlockSpec((1, D), lambda i: (0, 0)),
        ],
        out_specs=[
            pl.BlockSpec((BN, D), lambda i: (i, 0)),
            pl.BlockSpec((BN, D), lambda i: (i, 0)),
        ],
        out_shape=[
            jax.ShapeDtypeStruct((N, D), jnp.float32),
            jax.ShapeDtypeStruct((N, D), jnp.float32),
        ],
    )(x, h1, h2, p3, p3, dis_b, W, b)


def _finish2_body(x, h1, h2, pa, pb, dis, W, b, wf, y_ref):
    i = pl.program_id(0)
    x3 = _layer_out(x, h1, h2, pa, pb, dis, W, b)

    @pl.when(i == 0)
    def _():
        y_ref[...] = jnp.zeros((16, D), jnp.float32)

    rows = [jnp.sum(x3 * wf[g], axis=0, keepdims=True) for g in range(G)]
    rows.append(jnp.zeros((16 - G, D), jnp.float32))
    y_ref[...] = y_ref[...] + jnp.concatenate(rows, axis=0)


def _finish2(x, h1, h2, p3, dis_b, W, b, wfr):
    return pl.pallas_call(
        _finish2_body,
        grid=(NB,),
        in_specs=[
            pl.BlockSpec((BN, D), lambda i: (i, 0)),
            pl.BlockSpec((BN, D), lambda i: (i, 0)),
            pl.BlockSpec((BN, D), lambda i: (i, 0)),
            pl.BlockSpec((BN, D), lambda i: (i, 0)),
            pl.BlockSpec((BN, D), lambda i: (i + NB, 0)),
            pl.BlockSpec((BN, D), lambda i: (i, 0)),
            pl.BlockSpec((K + 1, D, D), lambda i: (0, 0, 0)),
            pl.BlockSpec((1, D), lambda i: (0, 0)),
            pl.BlockSpec((G, BN, D), lambda i: (0, i, 0)),
        ],
        out_specs=pl.BlockSpec((16, D), lambda i: (0, 0)),
        out_shape=jax.ShapeDtypeStruct((16, D), jnp.float32),
    )(x, h1, h2, p3, p3, dis_b, W, b, wfr)


def kernel(x, edge_index, W1, b1, W2, b2, Wf, bf):
    src = edge_index[0]
    dst = edge_index[1]
    wfr = Wf.reshape(G, N, D)
    b1r = b1.reshape(1, D)
    b2r = b2.reshape(1, D)

    degp = _degree(dst)
    dis_b, g = _prep(degp, x)

    h1, g = _combine(_seg_sum(src, dst, g), dis_b)
    h2, g = _combine(_seg_sum(src, dst, g), dis_b)
    p3 = _seg_sum(src, dst, g)
    x2, g = _finish1(x, h1, h2, p3, dis_b, W1, b1r)

    h1, g = _combine(_seg_sum(src, dst, g), dis_b)
    h2, g = _combine(_seg_sum(src, dst, g), dis_b)
    p3 = _seg_sum(src, dst, g)
    y16 = _finish2(x2, h1, h2, p3, dis_b, W2, b2r, wfr)

    return jnp.sum(y16[:G], axis=1) + bf


# speedup vs baseline: 2.7700x; 1.0116x over previous
import jax
import jax.numpy as jnp
from jax import lax
from jax.experimental import pallas as pl
from jax.experimental.pallas import tpu as pltpu
from jax.experimental.pallas import tpu_sc as plsc

N = 10000
E = 320000
D = 128
G = 10
K = 3

NC, NS = 2, 16
NW = NC * NS
CH = 128
NCHUNK = E // CH
BASE = NCHUNK // NW
EXTRA = NCHUNK - BASE * NW
NA = N
RPT = 624
RREM = N - NS * RPT
DEGW = 128

BN = 1000
NB = N // BN


def _zero_fill(buf, nrows, width):
    def row(i, _):
        for l in range(width // 16):
            buf[i, pl.ds(l * 16, 16)] = jnp.zeros((16,), jnp.float32)
        return 0
    lax.fori_loop(0, nrows, row, 0)


def _zero_acc(rows, acc, s):
    r0 = s * RPT
    for off, nr in ((0, 128), (128, 128), (256, 128), (384, 128), (512, 112)):
        pltpu.sync_copy(rows.at[pl.ds(0, nr)], acc.at[pl.ds(r0 + off, nr)])

    @pl.when(s == 0)
    def _():
        pltpu.sync_copy(rows.at[pl.ds(0, RREM)], acc.at[pl.ds(NS * RPT, RREM)])


def _publish(acc, out_hbm, c, s):
    r0 = s * RPT
    pltpu.sync_copy(acc.at[pl.ds(r0, RPT)], out_hbm.at[pl.ds(c * N + r0, RPT)])

    @pl.when(s == 0)
    def _():
        pltpu.sync_copy(acc.at[pl.ds(NS * RPT, RREM)],
                        out_hbm.at[pl.ds(c * N + NS * RPT, RREM)])


def _hop_body(ei_hbm, g_hbm, out_hbm,
              sidx0, sidx1, didx0, didx1, rows0, rows1, acc,
              semg0, semg1):
    c = lax.axis_index("c")
    s = lax.axis_index("s")
    wid = c * NS + s
    _zero_fill(rows0, CH, D)
    _zero_acc(rows0, acc, s)
    plsc.subcore_barrier()
    e0 = wid * BASE * CH

    pltpu.sync_copy(ei_hbm.at[0, pl.ds(e0, CH)], sidx0)
    pltpu.async_copy(g_hbm.at[sidx0], rows0, semg0)

    def pair(jj, _):
        j0 = 2 * jj
        pltpu.sync_copy(ei_hbm.at[0, pl.ds(e0 + (j0 + 1) * CH, CH)], sidx1)
        pltpu.async_copy(g_hbm.at[sidx1], rows1, semg1)
        pltpu.sync_copy(ei_hbm.at[1, pl.ds(e0 + j0 * CH, CH)], didx0)
        pltpu.make_async_copy(g_hbm.at[sidx0], rows0, semg0).wait()
        pltpu.sync_copy(rows0, acc.at[didx0], add=True)

        @pl.when(jj < BASE // 2 - 1)
        def _():
            pltpu.sync_copy(ei_hbm.at[0, pl.ds(e0 + (j0 + 2) * CH, CH)], sidx0)
            pltpu.async_copy(g_hbm.at[sidx0], rows0, semg0)
        pltpu.sync_copy(ei_hbm.at[1, pl.ds(e0 + (j0 + 1) * CH, CH)], didx1)
        pltpu.make_async_copy(g_hbm.at[sidx1], rows1, semg1).wait()
        pltpu.sync_copy(rows1, acc.at[didx1], add=True)
        return 0
    lax.fori_loop(0, BASE // 2, pair, 0)

    @pl.when(wid < EXTRA)
    def _():
        ex = (NW * BASE + wid) * CH
        pltpu.sync_copy(ei_hbm.at[0, pl.ds(ex, CH)], sidx0)
        pltpu.sync_copy(ei_hbm.at[1, pl.ds(ex, CH)], didx0)
        pltpu.async_copy(g_hbm.at[sidx0], rows0, semg0).wait()
        pltpu.sync_copy(rows0, acc.at[didx0], add=True)

    plsc.subcore_barrier()
    _publish(acc, out_hbm, c, s)


def _deg_body(ei_hbm, out_hbm, didx0, didx1, rows, acc, semd0, semd1):
    c = lax.axis_index("c")
    s = lax.axis_index("s")
    wid = c * NS + s
    _zero_fill(rows, CH, DEGW)
    _zero_acc(rows, acc, s)
    def row(i, _):
        for l in range(DEGW // 16):
            rows[i, pl.ds(l * 16, 16)] = jnp.ones((16,), jnp.float32)
        return 0
    lax.fori_loop(0, CH, row, 0)
    plsc.subcore_barrier()
    e0 = wid * BASE * CH

    pltpu.async_copy(ei_hbm.at[1, pl.ds(e0, CH)], didx0, semd0)
    pltpu.async_copy(ei_hbm.at[1, pl.ds(e0 + CH, CH)], didx1, semd1)

    def pair(jj, _):
        j0 = 2 * jj
        pltpu.make_async_copy(ei_hbm.at[1, pl.ds(e0, CH)], didx0, semd0).wait()
        pltpu.sync_copy(rows, acc.at[didx0], add=True)

        @pl.when(jj < BASE // 2 - 1)
        def _():
            pltpu.async_copy(
                ei_hbm.at[1, pl.ds(e0 + (j0 + 2) * CH, CH)], didx0, semd0)

        pltpu.make_async_copy(ei_hbm.at[1, pl.ds(e0, CH)], didx1, semd1).wait()
        pltpu.sync_copy(rows, acc.at[didx1], add=True)

        @pl.when(jj < BASE // 2 - 1)
        def _():
            pltpu.async_copy(
                ei_hbm.at[1, pl.ds(e0 + (j0 + 3) * CH, CH)], didx1, semd1)
        return 0
    lax.fori_loop(0, BASE // 2, pair, 0)

    @pl.when(wid < EXTRA)
    def _():
        ex = (NW * BASE + wid) * CH
        pltpu.sync_copy(ei_hbm.at[1, pl.ds(ex, CH)], didx0)
        pltpu.sync_copy(rows, acc.at[didx0], add=True)

    plsc.subcore_barrier()
    _publish(acc, out_hbm, c, s)


def _sc_mesh():
    return plsc.VectorSubcoreMesh(core_axis_name="c", subcore_axis_name="s",
                                  num_cores=NC, num_subcores=NS)


def _seg_sum(ei, g):
    return pl.kernel(
        _hop_body,
        out_type=jax.ShapeDtypeStruct((2 * N, D), jnp.float32),
        mesh=_sc_mesh(),
        scratch_types=[
            pltpu.VMEM((CH,), jnp.int32),
            pltpu.VMEM((CH,), jnp.int32),
            pltpu.VMEM((CH,), jnp.int32),
            pltpu.VMEM((CH,), jnp.int32),
            pltpu.VMEM((CH, D), jnp.float32),
            pltpu.VMEM((CH, D), jnp.float32),
            pltpu.VMEM_SHARED((NA, D), jnp.float32),
            pltpu.SemaphoreType.DMA,
            pltpu.SemaphoreType.DMA,
        ],
    )(ei, g)


def _degree(ei):
    return pl.kernel(
        _deg_body,
        out_type=jax.ShapeDtypeStruct((2 * N, DEGW), jnp.float32),
        mesh=_sc_mesh(),
        scratch_types=[
            pltpu.VMEM((CH,), jnp.int32),
            pltpu.VMEM((CH,), jnp.int32),
            pltpu.VMEM((CH, DEGW), jnp.float32),
            pltpu.VMEM_SHARED((NA, DEGW), jnp.float32),
            pltpu.SemaphoreType.DMA,
            pltpu.SemaphoreType.DMA,
        ],
    )(ei)


def _prep_body(p0, p1, x, dis_ref, g0_ref):
    deg = p0[:, 0:1] + p1[:, 0:1]
    dis = jnp.where(deg > 0, lax.rsqrt(jnp.maximum(deg, 1.0)), 0.0)
    dis_b = jnp.broadcast_to(dis, (BN, D))
    dis_ref[...] = dis_b
    g0_ref[...] = dis_b * x[...]


def _prep(degp, x):
    return pl.pallas_call(
        _prep_body,
        grid=(NB,),
        in_specs=[
            pl.BlockSpec((BN, DEGW), lambda i: (i, 0)),
            pl.BlockSpec((BN, DEGW), lambda i: (i + NB, 0)),
            pl.BlockSpec((BN, D), lambda i: (i, 0)),
        ],
        out_specs=[
            pl.BlockSpec((BN, D), lambda i: (i, 0)),
            pl.BlockSpec((BN, D), lambda i: (i, 0)),
        ],
        out_shape=[
            jax.ShapeDtypeStruct((N, D), jnp.float32),
            jax.ShapeDtypeStruct((N, D), jnp.float32),
        ],
    )(degp, degp, x)


def _combine_body(pa, pb, dis, h_ref, g_ref):
    h = dis[...] * (pa[...] + pb[...])
    h_ref[...] = h
    g_ref[...] = dis[...] * h


def _combine(p, dis_b):
    return pl.pallas_call(
        _combine_body,
        grid=(NB,),
        in_specs=[
            pl.BlockSpec((BN, D), lambda i: (i, 0)),
            pl.BlockSpec((BN, D), lambda i: (i + NB, 0)),
            pl.BlockSpec((BN, D), lambda i: (i, 0)),
        ],
        out_specs=[
            pl.BlockSpec((BN, D), lambda i: (i, 0)),
            pl.BlockSpec((BN, D), lambda i: (i, 0)),
        ],
        out_shape=[
            jax.ShapeDtypeStruct((N, D), jnp.float32),
            jax.ShapeDtypeStruct((N, D), jnp.float32),
        ],
    )(p, p, dis_b)


def _layer_out(x, h1, h2, pa, pb, dis, W, b):
    h3 = dis[...] * (pa[...] + pb[...])
    acc = jnp.dot(x[...], W[0], preferred_element_type=jnp.float32)
    acc = acc + jnp.dot(h1[...], W[1], preferred_element_type=jnp.float32)
    acc = acc + jnp.dot(h2[...], W[2], preferred_element_type=jnp.float32)
    acc = acc + jnp.dot(h3, W[3], preferred_element_type=jnp.float32)
    acc = acc + b[...]
    return jnp.where(acc >= 0, acc, 0.01 * acc)


def _finish1_body(x, h1, h2, pa, pb, dis, W, b, x2_ref, g_ref):
    x2 = _layer_out(x, h1, h2, pa, pb, dis, W, b)
    x2_ref[...] = x2
    g_ref[...] = dis[...] * x2


def _finish1(x, h1, h2, p3, dis_b, W, b):
    return pl.pallas_call(
        _finish1_body,
        grid=(NB,),
        in_specs=[
            pl.BlockSpec((BN, D), lambda i: (i, 0)),
            pl.BlockSpec((BN, D), lambda i: (i, 0)),
            pl.BlockSpec((BN, D), lambda i: (i, 0)),
            pl.BlockSpec((BN, D), lambda i: (i, 0)),
            pl.BlockSpec((BN, D), lambda i: (i + NB, 0)),
            pl.BlockSpec((BN, D), lambda i: (i, 0)),
            pl.BlockSpec((K + 1, D, D), lambda i: (0, 0, 0)),
            pl.BlockSpec((1, D), lambda i: (0, 0)),
        ],
        out_specs=[
            pl.BlockSpec((BN, D), lambda i: (i, 0)),
            pl.BlockSpec((BN, D), lambda i: (i, 0)),
        ],
        out_shape=[
            jax.ShapeDtypeStruct((N, D), jnp.float32),
            jax.ShapeDtypeStruct((N, D), jnp.float32),
        ],
    )(x, h1, h2, p3, p3, dis_b, W, b)


def _finish2_body(x, h1, h2, pa, pb, dis, W, b, wf, y_ref):
    i = pl.program_id(0)
    x3 = _layer_out(x, h1, h2, pa, pb, dis, W, b)

    @pl.when(i == 0)
    def _():
        y_ref[...] = jnp.zeros((16, D), jnp.float32)

    rows = [jnp.sum(x3 * wf[g], axis=0, keepdims=True) for g in range(G)]
    rows.append(jnp.zeros((16 - G, D), jnp.float32))
    y_ref[...] = y_ref[...] + jnp.concatenate(rows, axis=0)


def _finish2(x, h1, h2, p3, dis_b, W, b, wfr):
    return pl.pallas_call(
        _finish2_body,
        grid=(NB,),
        in_specs=[
            pl.BlockSpec((BN, D), lambda i: (i, 0)),
            pl.BlockSpec((BN, D), lambda i: (i, 0)),
            pl.BlockSpec((BN, D), lambda i: (i, 0)),
            pl.BlockSpec((BN, D), lambda i: (i, 0)),
            pl.BlockSpec((BN, D), lambda i: (i + NB, 0)),
            pl.BlockSpec((BN, D), lambda i: (i, 0)),
            pl.BlockSpec((K + 1, D, D), lambda i: (0, 0, 0)),
            pl.BlockSpec((1, D), lambda i: (0, 0)),
            pl.BlockSpec((G, BN, D), lambda i: (0, i, 0)),
        ],
        out_specs=pl.BlockSpec((16, D), lambda i: (0, 0)),
        out_shape=jax.ShapeDtypeStruct((16, D), jnp.float32),
    )(x, h1, h2, p3, p3, dis_b, W, b, wfr)


def kernel(x, edge_index, W1, b1, W2, b2, Wf, bf):
    wfr = Wf.reshape(G, N, D)
    b1r = b1.reshape(1, D)
    b2r = b2.reshape(1, D)

    degp = _degree(edge_index)
    dis_b, g = _prep(degp, x)

    h1, g = _combine(_seg_sum(edge_index, g), dis_b)
    h2, g = _combine(_seg_sum(edge_index, g), dis_b)
    p3 = _seg_sum(edge_index, g)
    x2, g = _finish1(x, h1, h2, p3, dis_b, W1, b1r)

    h1, g = _combine(_seg_sum(edge_index, g), dis_b)
    h2, g = _combine(_seg_sum(edge_index, g), dis_b)
    p3 = _seg_sum(edge_index, g)
    y16 = _finish2(x2, h1, h2, p3, dis_b, W2, b2r, wfr)

    return jnp.sum(y16[:G], axis=1) + bf


# speedup vs baseline: 3.1882x; 1.1510x over previous
import jax
import jax.numpy as jnp
from jax import lax
from jax.experimental import pallas as pl
from jax.experimental.pallas import tpu as pltpu
from jax.experimental.pallas import tpu_sc as plsc

N = 10000
E = 320000
D = 128
G = 10
K = 3

NC, NS = 2, 16
NW = NC * NS
CH = 128
NCHUNK = E // CH
BASE = NCHUNK // NW
EXTRA = NCHUNK - BASE * NW
NA = N
RPT = 624
RREM = N - NS * RPT
DEGW = 128

BN = 1000
NB = N // BN


def _zero_fill(buf, nrows, width):
    def row(i, _):
        for l in range(width // 16):
            buf[i, pl.ds(l * 16, 16)] = jnp.zeros((16,), jnp.float32)
        return 0
    lax.fori_loop(0, nrows, row, 0)


def _zero_acc(rows, acc, s):
    r0 = s * RPT
    for off, nr in ((0, 128), (128, 128), (256, 128), (384, 128), (512, 112)):
        pltpu.sync_copy(rows.at[pl.ds(0, nr)], acc.at[pl.ds(r0 + off, nr)])

    @pl.when(s == 0)
    def _():
        pltpu.sync_copy(rows.at[pl.ds(0, RREM)], acc.at[pl.ds(NS * RPT, RREM)])


def _publish(acc, out_hbm, c, s):
    r0 = s * RPT
    pltpu.sync_copy(acc.at[pl.ds(r0, RPT)], out_hbm.at[pl.ds(c * N + r0, RPT)])

    @pl.when(s == 0)
    def _():
        pltpu.sync_copy(acc.at[pl.ds(NS * RPT, RREM)],
                        out_hbm.at[pl.ds(c * N + NS * RPT, RREM)])


def _hop_body(ei_hbm, g_hbm, out_hbm,
              sidx0, sidx1, didx0, didx1, rows0, rows1, acc,
              semg0, semg1, semd0, semd1):
    c = lax.axis_index("c")
    s = lax.axis_index("s")
    wid = c * NS + s
    _zero_fill(rows0, CH, D)
    _zero_acc(rows0, acc, s)
    plsc.subcore_barrier()
    e0 = wid * BASE * CH

    pltpu.sync_copy(ei_hbm.at[0, pl.ds(e0, CH)], sidx0)
    pltpu.async_copy(g_hbm.at[sidx0], rows0, semg0)
    pltpu.async_copy(ei_hbm.at[1, pl.ds(e0, CH)], didx0, semd0)
    pltpu.async_copy(ei_hbm.at[1, pl.ds(e0 + CH, CH)], didx1, semd1)

    def pair(jj, _):
        j0 = 2 * jj
        pltpu.sync_copy(ei_hbm.at[0, pl.ds(e0 + (j0 + 1) * CH, CH)], sidx1)
        pltpu.async_copy(g_hbm.at[sidx1], rows1, semg1)
        pltpu.make_async_copy(ei_hbm.at[1, pl.ds(e0, CH)], didx0, semd0).wait()
        pltpu.make_async_copy(g_hbm.at[sidx0], rows0, semg0).wait()
        pltpu.sync_copy(rows0, acc.at[didx0], add=True)

        @pl.when(jj < BASE // 2 - 1)
        def _():
            pltpu.sync_copy(ei_hbm.at[0, pl.ds(e0 + (j0 + 2) * CH, CH)], sidx0)
            pltpu.async_copy(g_hbm.at[sidx0], rows0, semg0)
            pltpu.async_copy(
                ei_hbm.at[1, pl.ds(e0 + (j0 + 2) * CH, CH)], didx0, semd0)
        pltpu.make_async_copy(ei_hbm.at[1, pl.ds(e0, CH)], didx1, semd1).wait()
        pltpu.make_async_copy(g_hbm.at[sidx1], rows1, semg1).wait()
        pltpu.sync_copy(rows1, acc.at[didx1], add=True)

        @pl.when(jj < BASE // 2 - 1)
        def _():
            pltpu.async_copy(
                ei_hbm.at[1, pl.ds(e0 + (j0 + 3) * CH, CH)], didx1, semd1)
        return 0
    lax.fori_loop(0, BASE // 2, pair, 0)

    @pl.when(wid < EXTRA)
    def _():
        ex = (NW * BASE + wid) * CH
        pltpu.sync_copy(ei_hbm.at[0, pl.ds(ex, CH)], sidx0)
        pltpu.sync_copy(ei_hbm.at[1, pl.ds(ex, CH)], didx0)
        pltpu.async_copy(g_hbm.at[sidx0], rows0, semg0).wait()
        pltpu.sync_copy(rows0, acc.at[didx0], add=True)

    plsc.subcore_barrier()
    _publish(acc, out_hbm, c, s)


def _deg_body(ei_hbm, out_hbm, didx0, didx1, rows, acc, semd0, semd1):
    c = lax.axis_index("c")
    s = lax.axis_index("s")
    wid = c * NS + s
    _zero_fill(rows, CH, DEGW)
    _zero_acc(rows, acc, s)
    def row(i, _):
        for l in range(DEGW // 16):
            rows[i, pl.ds(l * 16, 16)] = jnp.ones((16,), jnp.float32)
        return 0
    lax.fori_loop(0, CH, row, 0)
    plsc.subcore_barrier()
    e0 = wid * BASE * CH

    pltpu.async_copy(ei_hbm.at[1, pl.ds(e0, CH)], didx0, semd0)
    pltpu.async_copy(ei_hbm.at[1, pl.ds(e0 + CH, CH)], didx1, semd1)

    def pair(jj, _):
        j0 = 2 * jj
        pltpu.make_async_copy(ei_hbm.at[1, pl.ds(e0, CH)], didx0, semd0).wait()
        pltpu.sync_copy(rows, acc.at[didx0], add=True)

        @pl.when(jj < BASE // 2 - 1)
        def _():
            pltpu.async_copy(
                ei_hbm.at[1, pl.ds(e0 + (j0 + 2) * CH, CH)], didx0, semd0)

        pltpu.make_async_copy(ei_hbm.at[1, pl.ds(e0, CH)], didx1, semd1).wait()
        pltpu.sync_copy(rows, acc.at[didx1], add=True)

        @pl.when(jj < BASE // 2 - 1)
        def _():
            pltpu.async_copy(
                ei_hbm.at[1, pl.ds(e0 + (j0 + 3) * CH, CH)], didx1, semd1)
        return 0
    lax.fori_loop(0, BASE // 2, pair, 0)

    @pl.when(wid < EXTRA)
    def _():
        ex = (NW * BASE + wid) * CH
        pltpu.sync_copy(ei_hbm.at[1, pl.ds(ex, CH)], didx0)
        pltpu.sync_copy(rows, acc.at[didx0], add=True)

    plsc.subcore_barrier()
    _publish(acc, out_hbm, c, s)


def _sc_mesh():
    return plsc.VectorSubcoreMesh(core_axis_name="c", subcore_axis_name="s",
                                  num_cores=NC, num_subcores=NS)


def _seg_sum(ei, g):
    return pl.kernel(
        _hop_body,
        out_type=jax.ShapeDtypeStruct((2 * N, D), jnp.float32),
        mesh=_sc_mesh(),
        scratch_types=[
            pltpu.VMEM((CH,), jnp.int32),
            pltpu.VMEM((CH,), jnp.int32),
            pltpu.VMEM((CH,), jnp.int32),
            pltpu.VMEM((CH,), jnp.int32),
            pltpu.VMEM((CH, D), jnp.float32),
            pltpu.VMEM((CH, D), jnp.float32),
            pltpu.VMEM_SHARED((NA, D), jnp.float32),
            pltpu.SemaphoreType.DMA,
            pltpu.SemaphoreType.DMA,
            pltpu.SemaphoreType.DMA,
            pltpu.SemaphoreType.DMA,
        ],
    )(ei, g)


def _degree(ei):
    return pl.kernel(
        _deg_body,
        out_type=jax.ShapeDtypeStruct((2 * N, DEGW), jnp.float32),
        mesh=_sc_mesh(),
        scratch_types=[
            pltpu.VMEM((CH,), jnp.int32),
            pltpu.VMEM((CH,), jnp.int32),
            pltpu.VMEM((CH, DEGW), jnp.float32),
            pltpu.VMEM_SHARED((NA, DEGW), jnp.float32),
            pltpu.SemaphoreType.DMA,
            pltpu.SemaphoreType.DMA,
        ],
    )(ei)


def _prep_body(p0, p1, x, dis_ref, g0_ref):
    deg = p0[:, 0:1] + p1[:, 0:1]
    dis = jnp.where(deg > 0, lax.rsqrt(jnp.maximum(deg, 1.0)), 0.0)
    dis_b = jnp.broadcast_to(dis, (BN, D))
    dis_ref[...] = dis_b
    g0_ref[...] = dis_b * x[...]


def _prep(degp, x):
    return pl.pallas_call(
        _prep_body,
        grid=(NB,),
        in_specs=[
            pl.BlockSpec((BN, DEGW), lambda i: (i, 0)),
            pl.BlockSpec((BN, DEGW), lambda i: (i + NB, 0)),
            pl.BlockSpec((BN, D), lambda i: (i, 0)),
        ],
        out_specs=[
            pl.BlockSpec((BN, D), lambda i: (i, 0)),
            pl.BlockSpec((BN, D), lambda i: (i, 0)),
        ],
        out_shape=[
            jax.ShapeDtypeStruct((N, D), jnp.float32),
            jax.ShapeDtypeStruct((N, D), jnp.float32),
        ],
    )(degp, degp, x)


def _combine_body(pa, pb, dis, h_ref, g_ref):
    h = dis[...] * (pa[...] + pb[...])
    h_ref[...] = h
    g_ref[...] = dis[...] * h


def _combine(p, dis_b):
    return pl.pallas_call(
        _combine_body,
        grid=(NB,),
        in_specs=[
            pl.BlockSpec((BN, D), lambda i: (i, 0)),
            pl.BlockSpec((BN, D), lambda i: (i + NB, 0)),
            pl.BlockSpec((BN, D), lambda i: (i, 0)),
        ],
        out_specs=[
            pl.BlockSpec((BN, D), lambda i: (i, 0)),
            pl.BlockSpec((BN, D), lambda i: (i, 0)),
        ],
        out_shape=[
            jax.ShapeDtypeStruct((N, D), jnp.float32),
            jax.ShapeDtypeStruct((N, D), jnp.float32),
        ],
    )(p, p, dis_b)


def _layer_out(x, h1, h2, pa, pb, dis, W, b):
    h3 = dis[...] * (pa[...] + pb[...])
    acc = jnp.dot(x[...], W[0], preferred_element_type=jnp.float32)
    acc = acc + jnp.dot(h1[...], W[1], preferred_element_type=jnp.float32)
    acc = acc + jnp.dot(h2[...], W[2], preferred_element_type=jnp.float32)
    acc = acc + jnp.dot(h3, W[3], preferred_element_type=jnp.float32)
    acc = acc + b[...]
    return jnp.where(acc >= 0, acc, 0.01 * acc)


def _finish1_body(x, h1, h2, pa, pb, dis, W, b, x2_ref, g_ref):
    x2 = _layer_out(x, h1, h2, pa, pb, dis, W, b)
    x2_ref[...] = x2
    g_ref[...] = dis[...] * x2


def _finish1(x, h1, h2, p3, dis_b, W, b):
    return pl.pallas_call(
        _finish1_body,
        grid=(NB,),
        in_specs=[
            pl.BlockSpec((BN, D), lambda i: (i, 0)),
            pl.BlockSpec((BN, D), lambda i: (i, 0)),
            pl.BlockSpec((BN, D), lambda i: (i, 0)),
            pl.BlockSpec((BN, D), lambda i: (i, 0)),
            pl.BlockSpec((BN, D), lambda i: (i + NB, 0)),
            pl.BlockSpec((BN, D), lambda i: (i, 0)),
            pl.BlockSpec((K + 1, D, D), lambda i: (0, 0, 0)),
            pl.BlockSpec((1, D), lambda i: (0, 0)),
        ],
        out_specs=[
            pl.BlockSpec((BN, D), lambda i: (i, 0)),
            pl.BlockSpec((BN, D), lambda i: (i, 0)),
        ],
        out_shape=[
            jax.ShapeDtypeStruct((N, D), jnp.float32),
            jax.ShapeDtypeStruct((N, D), jnp.float32),
        ],
    )(x, h1, h2, p3, p3, dis_b, W, b)


def _finish2_body(x, h1, h2, pa, pb, dis, W, b, wf, y_ref):
    i = pl.program_id(0)
    x3 = _layer_out(x, h1, h2, pa, pb, dis, W, b)

    @pl.when(i == 0)
    def _():
        y_ref[...] = jnp.zeros((16, D), jnp.float32)

    rows = [jnp.sum(x3 * wf[g], axis=0, keepdims=True) for g in range(G)]
    rows.append(jnp.zeros((16 - G, D), jnp.float32))
    y_ref[...] = y_ref[...] + jnp.concatenate(rows, axis=0)


def _finish2(x, h1, h2, p3, dis_b, W, b, wfr):
    return pl.pallas_call(
        _finish2_body,
        grid=(NB,),
        in_specs=[
            pl.BlockSpec((BN, D), lambda i: (i, 0)),
            pl.BlockSpec((BN, D), lambda i: (i, 0)),
            pl.BlockSpec((BN, D), lambda i: (i, 0)),
            pl.BlockSpec((BN, D), lambda i: (i, 0)),
            pl.BlockSpec((BN, D), lambda i: (i + NB, 0)),
            pl.BlockSpec((BN, D), lambda i: (i, 0)),
            pl.BlockSpec((K + 1, D, D), lambda i: (0, 0, 0)),
            pl.BlockSpec((1, D), lambda i: (0, 0)),
            pl.BlockSpec((G, BN, D), lambda i: (0, i, 0)),
        ],
        out_specs=pl.BlockSpec((16, D), lambda i: (0, 0)),
        out_shape=jax.ShapeDtypeStruct((16, D), jnp.float32),
    )(x, h1, h2, p3, p3, dis_b, W, b, wfr)


def kernel(x, edge_index, W1, b1, W2, b2, Wf, bf):
    wfr = Wf.reshape(G, N, D)
    b1r = b1.reshape(1, D)
    b2r = b2.reshape(1, D)

    degp = _degree(edge_index)
    dis_b, g = _prep(degp, x)

    h1, g = _combine(_seg_sum(edge_index, g), dis_b)
    h2, g = _combine(_seg_sum(edge_index, g), dis_b)
    p3 = _seg_sum(edge_index, g)
    x2, g = _finish1(x, h1, h2, p3, dis_b, W1, b1r)

    h1, g = _combine(_seg_sum(edge_index, g), dis_b)
    h2, g = _combine(_seg_sum(edge_index, g), dis_b)
    p3 = _seg_sum(edge_index, g)
    y16 = _finish2(x2, h1, h2, p3, dis_b, W2, b2r, wfr)

    return jnp.sum(y16[:G], axis=1) + bf


# speedup vs baseline: 3.1902x; 1.0006x over previous
import jax
import jax.numpy as jnp
from jax import lax
from jax.experimental import pallas as pl
from jax.experimental.pallas import tpu as pltpu
from jax.experimental.pallas import tpu_sc as plsc

N = 10000
E = 320000
D = 128
G = 10
K = 3

NC, NS = 2, 16
NW = NC * NS
CH = 128
NCHUNK = E // CH
BASE = NCHUNK // NW
EXTRA = NCHUNK - BASE * NW
NA = N
RPT = 624
RREM = N - NS * RPT
DEGW = 128

BN = 1000
NB = N // BN


def _zero_fill(buf, nrows, width):
    def row(i, _):
        for l in range(width // 16):
            buf[i, pl.ds(l * 16, 16)] = jnp.zeros((16,), jnp.float32)
        return 0
    lax.fori_loop(0, nrows, row, 0)


def _zero_acc(rows, acc, s):
    r0 = s * RPT
    for off, nr in ((0, 128), (128, 128), (256, 128), (384, 128), (512, 112)):
        pltpu.sync_copy(rows.at[pl.ds(0, nr)], acc.at[pl.ds(r0 + off, nr)])

    @pl.when(s == 0)
    def _():
        pltpu.sync_copy(rows.at[pl.ds(0, RREM)], acc.at[pl.ds(NS * RPT, RREM)])


def _publish(acc, out_hbm, c, s):
    r0 = s * RPT
    pltpu.sync_copy(acc.at[pl.ds(r0, RPT)], out_hbm.at[pl.ds(c * N + r0, RPT)])

    @pl.when(s == 0)
    def _():
        pltpu.sync_copy(acc.at[pl.ds(NS * RPT, RREM)],
                        out_hbm.at[pl.ds(c * N + NS * RPT, RREM)])


def _hop_body(ei_hbm, g_hbm, out_hbm,
              sidx0, sidx1, didx0, didx1, rows0, rows1, acc,
              semg0, semg1, semd0, semd1, sems1):
    c = lax.axis_index("c")
    s = lax.axis_index("s")
    wid = c * NS + s
    _zero_fill(rows0, CH, D)
    _zero_acc(rows0, acc, s)
    plsc.subcore_barrier()
    e0 = wid * BASE * CH

    pltpu.sync_copy(ei_hbm.at[0, pl.ds(e0, CH)], sidx0)
    pltpu.async_copy(g_hbm.at[sidx0], rows0, semg0)
    pltpu.async_copy(ei_hbm.at[0, pl.ds(e0 + CH, CH)], sidx1, sems1)
    pltpu.async_copy(ei_hbm.at[1, pl.ds(e0, CH)], didx0, semd0)
    pltpu.async_copy(ei_hbm.at[1, pl.ds(e0 + CH, CH)], didx1, semd1)

    def pair(jj, _):
        j0 = 2 * jj
        pltpu.make_async_copy(ei_hbm.at[0, pl.ds(e0, CH)], sidx1, sems1).wait()
        pltpu.async_copy(g_hbm.at[sidx1], rows1, semg1)
        pltpu.make_async_copy(ei_hbm.at[1, pl.ds(e0, CH)], didx0, semd0).wait()
        pltpu.make_async_copy(g_hbm.at[sidx0], rows0, semg0).wait()
        pltpu.sync_copy(rows0, acc.at[didx0], add=True)

        @pl.when(jj < BASE // 2 - 1)
        def _():
            pltpu.sync_copy(ei_hbm.at[0, pl.ds(e0 + (j0 + 2) * CH, CH)], sidx0)
            pltpu.async_copy(g_hbm.at[sidx0], rows0, semg0)
            pltpu.async_copy(
                ei_hbm.at[1, pl.ds(e0 + (j0 + 2) * CH, CH)], didx0, semd0)
        pltpu.make_async_copy(ei_hbm.at[1, pl.ds(e0, CH)], didx1, semd1).wait()
        pltpu.make_async_copy(g_hbm.at[sidx1], rows1, semg1).wait()
        pltpu.sync_copy(rows1, acc.at[didx1], add=True)

        @pl.when(jj < BASE // 2 - 1)
        def _():
            pltpu.async_copy(
                ei_hbm.at[0, pl.ds(e0 + (j0 + 3) * CH, CH)], sidx1, sems1)
            pltpu.async_copy(
                ei_hbm.at[1, pl.ds(e0 + (j0 + 3) * CH, CH)], didx1, semd1)
        return 0
    lax.fori_loop(0, BASE // 2, pair, 0)

    @pl.when(wid < EXTRA)
    def _():
        ex = (NW * BASE + wid) * CH
        pltpu.sync_copy(ei_hbm.at[0, pl.ds(ex, CH)], sidx0)
        pltpu.sync_copy(ei_hbm.at[1, pl.ds(ex, CH)], didx0)
        pltpu.async_copy(g_hbm.at[sidx0], rows0, semg0).wait()
        pltpu.sync_copy(rows0, acc.at[didx0], add=True)

    plsc.subcore_barrier()
    _publish(acc, out_hbm, c, s)


def _deg_body(ei_hbm, out_hbm, didx0, didx1, rows, acc, semd0, semd1):
    c = lax.axis_index("c")
    s = lax.axis_index("s")
    wid = c * NS + s
    _zero_fill(rows, CH, DEGW)
    _zero_acc(rows, acc, s)
    def row(i, _):
        for l in range(DEGW // 16):
            rows[i, pl.ds(l * 16, 16)] = jnp.ones((16,), jnp.float32)
        return 0
    lax.fori_loop(0, CH, row, 0)
    plsc.subcore_barrier()
    e0 = wid * BASE * CH

    pltpu.async_copy(ei_hbm.at[1, pl.ds(e0, CH)], didx0, semd0)
    pltpu.async_copy(ei_hbm.at[1, pl.ds(e0 + CH, CH)], didx1, semd1)

    def pair(jj, _):
        j0 = 2 * jj
        pltpu.make_async_copy(ei_hbm.at[1, pl.ds(e0, CH)], didx0, semd0).wait()
        pltpu.sync_copy(rows, acc.at[didx0], add=True)

        @pl.when(jj < BASE // 2 - 1)
        def _():
            pltpu.async_copy(
                ei_hbm.at[1, pl.ds(e0 + (j0 + 2) * CH, CH)], didx0, semd0)

        pltpu.make_async_copy(ei_hbm.at[1, pl.ds(e0, CH)], didx1, semd1).wait()
        pltpu.sync_copy(rows, acc.at[didx1], add=True)

        @pl.when(jj < BASE // 2 - 1)
        def _():
            pltpu.async_copy(
                ei_hbm.at[1, pl.ds(e0 + (j0 + 3) * CH, CH)], didx1, semd1)
        return 0
    lax.fori_loop(0, BASE // 2, pair, 0)

    @pl.when(wid < EXTRA)
    def _():
        ex = (NW * BASE + wid) * CH
        pltpu.sync_copy(ei_hbm.at[1, pl.ds(ex, CH)], didx0)
        pltpu.sync_copy(rows, acc.at[didx0], add=True)

    plsc.subcore_barrier()
    _publish(acc, out_hbm, c, s)


def _sc_mesh():
    return plsc.VectorSubcoreMesh(core_axis_name="c", subcore_axis_name="s",
                                  num_cores=NC, num_subcores=NS)


def _seg_sum(ei, g):
    return pl.kernel(
        _hop_body,
        out_type=jax.ShapeDtypeStruct((2 * N, D), jnp.float32),
        mesh=_sc_mesh(),
        scratch_types=[
            pltpu.VMEM((CH,), jnp.int32),
            pltpu.VMEM((CH,), jnp.int32),
            pltpu.VMEM((CH,), jnp.int32),
            pltpu.VMEM((CH,), jnp.int32),
            pltpu.VMEM((CH, D), jnp.float32),
            pltpu.VMEM((CH, D), jnp.float32),
            pltpu.VMEM_SHARED((NA, D), jnp.float32),
            pltpu.SemaphoreType.DMA,
            pltpu.SemaphoreType.DMA,
            pltpu.SemaphoreType.DMA,
            pltpu.SemaphoreType.DMA,
            pltpu.SemaphoreType.DMA,
        ],
    )(ei, g)


def _degree(ei):
    return pl.kernel(
        _deg_body,
        out_type=jax.ShapeDtypeStruct((2 * N, DEGW), jnp.float32),
        mesh=_sc_mesh(),
        scratch_types=[
            pltpu.VMEM((CH,), jnp.int32),
            pltpu.VMEM((CH,), jnp.int32),
            pltpu.VMEM((CH, DEGW), jnp.float32),
            pltpu.VMEM_SHARED((NA, DEGW), jnp.float32),
            pltpu.SemaphoreType.DMA,
            pltpu.SemaphoreType.DMA,
        ],
    )(ei)


def _prep_body(p0, p1, x, dis_ref, g0_ref):
    deg = p0[:, 0:1] + p1[:, 0:1]
    dis = jnp.where(deg > 0, lax.rsqrt(jnp.maximum(deg, 1.0)), 0.0)
    dis_b = jnp.broadcast_to(dis, (BN, D))
    dis_ref[...] = dis_b
    g0_ref[...] = dis_b * x[...]


def _prep(degp, x):
    return pl.pallas_call(
        _prep_body,
        grid=(NB,),
        in_specs=[
            pl.BlockSpec((BN, DEGW), lambda i: (i, 0)),
            pl.BlockSpec((BN, DEGW), lambda i: (i + NB, 0)),
            pl.BlockSpec((BN, D), lambda i: (i, 0)),
        ],
        out_specs=[
            pl.BlockSpec((BN, D), lambda i: (i, 0)),
            pl.BlockSpec((BN, D), lambda i: (i, 0)),
        ],
        out_shape=[
            jax.ShapeDtypeStruct((N, D), jnp.float32),
            jax.ShapeDtypeStruct((N, D), jnp.float32),
        ],
    )(degp, degp, x)


def _combine_body(pa, pb, dis, h_ref, g_ref):
    h = dis[...] * (pa[...] + pb[...])
    h_ref[...] = h
    g_ref[...] = dis[...] * h


def _combine(p, dis_b):
    return pl.pallas_call(
        _combine_body,
        grid=(NB,),
        in_specs=[
            pl.BlockSpec((BN, D), lambda i: (i, 0)),
            pl.BlockSpec((BN, D), lambda i: (i + NB, 0)),
            pl.BlockSpec((BN, D), lambda i: (i, 0)),
        ],
        out_specs=[
            pl.BlockSpec((BN, D), lambda i: (i, 0)),
            pl.BlockSpec((BN, D), lambda i: (i, 0)),
        ],
        out_shape=[
            jax.ShapeDtypeStruct((N, D), jnp.float32),
            jax.ShapeDtypeStruct((N, D), jnp.float32),
        ],
    )(p, p, dis_b)


def _layer_out(x, h1, h2, pa, pb, dis, W, b):
    h3 = dis[...] * (pa[...] + pb[...])
    acc = jnp.dot(x[...], W[0], preferred_element_type=jnp.float32)
    acc = acc + jnp.dot(h1[...], W[1], preferred_element_type=jnp.float32)
    acc = acc + jnp.dot(h2[...], W[2], preferred_element_type=jnp.float32)
    acc = acc + jnp.dot(h3, W[3], preferred_element_type=jnp.float32)
    acc = acc + b[...]
    return jnp.where(acc >= 0, acc, 0.01 * acc)


def _finish1_body(x, h1, h2, pa, pb, dis, W, b, x2_ref, g_ref):
    x2 = _layer_out(x, h1, h2, pa, pb, dis, W, b)
    x2_ref[...] = x2
    g_ref[...] = dis[...] * x2


def _finish1(x, h1, h2, p3, dis_b, W, b):
    return pl.pallas_call(
        _finish1_body,
        grid=(NB,),
        in_specs=[
            pl.BlockSpec((BN, D), lambda i: (i, 0)),
            pl.BlockSpec((BN, D), lambda i: (i, 0)),
            pl.BlockSpec((BN, D), lambda i: (i, 0)),
            pl.BlockSpec((BN, D), lambda i: (i, 0)),
            pl.BlockSpec((BN, D), lambda i: (i + NB, 0)),
            pl.BlockSpec((BN, D), lambda i: (i, 0)),
            pl.BlockSpec((K + 1, D, D), lambda i: (0, 0, 0)),
            pl.BlockSpec((1, D), lambda i: (0, 0)),
        ],
        out_specs=[
            pl.BlockSpec((BN, D), lambda i: (i, 0)),
            pl.BlockSpec((BN, D), lambda i: (i, 0)),
        ],
        out_shape=[
            jax.ShapeDtypeStruct((N, D), jnp.float32),
            jax.ShapeDtypeStruct((N, D), jnp.float32),
        ],
    )(x, h1, h2, p3, p3, dis_b, W, b)


def _finish2_body(x, h1, h2, pa, pb, dis, W, b, wf, y_ref):
    i = pl.program_id(0)
    x3 = _layer_out(x, h1, h2, pa, pb, dis, W, b)

    @pl.when(i == 0)
    def _():
        y_ref[...] = jnp.zeros((16, D), jnp.float32)

    rows = [jnp.sum(x3 * wf[g], axis=0, keepdims=True) for g in range(G)]
    rows.append(jnp.zeros((16 - G, D), jnp.float32))
    y_ref[...] = y_ref[...] + jnp.concatenate(rows, axis=0)


def _finish2(x, h1, h2, p3, dis_b, W, b, wfr):
    return pl.pallas_call(
        _finish2_body,
        grid=(NB,),
        in_specs=[
            pl.BlockSpec((BN, D), lambda i: (i, 0)),
            pl.BlockSpec((BN, D), lambda i: (i, 0)),
            pl.BlockSpec((BN, D), lambda i: (i, 0)),
            pl.BlockSpec((BN, D), lambda i: (i, 0)),
            pl.BlockSpec((BN, D), lambda i: (i + NB, 0)),
            pl.BlockSpec((BN, D), lambda i: (i, 0)),
            pl.BlockSpec((K + 1, D, D), lambda i: (0, 0, 0)),
            pl.BlockSpec((1, D), lambda i: (0, 0)),
            pl.BlockSpec((G, BN, D), lambda i: (0, i, 0)),
        ],
        out_specs=pl.BlockSpec((16, D), lambda i: (0, 0)),
        out_shape=jax.ShapeDtypeStruct((16, D), jnp.float32),
    )(x, h1, h2, p3, p3, dis_b, W, b, wfr)


def kernel(x, edge_index, W1, b1, W2, b2, Wf, bf):
    wfr = Wf.reshape(G, N, D)
    b1r = b1.reshape(1, D)
    b2r = b2.reshape(1, D)

    degp = _degree(edge_index)
    dis_b, g = _prep(degp, x)

    h1, g = _combine(_seg_sum(edge_index, g), dis_b)
    h2, g = _combine(_seg_sum(edge_index, g), dis_b)
    p3 = _seg_sum(edge_index, g)
    x2, g = _finish1(x, h1, h2, p3, dis_b, W1, b1r)

    h1, g = _combine(_seg_sum(edge_index, g), dis_b)
    h2, g = _combine(_seg_sum(edge_index, g), dis_b)
    p3 = _seg_sum(edge_index, g)
    y16 = _finish2(x2, h1, h2, p3, dis_b, W2, b2r, wfr)

    return jnp.sum(y16[:G], axis=1) + bf
